# Initial kernel scaffold; baseline (speedup 1.0000x reference)
import jax
import jax.numpy as jnp
from jax.experimental import pallas as pl


def kernel(x, table, gamma, beta):
    raise NotImplementedError("write your pallas kernel here")


# speedup vs baseline: 1.9946x; 1.9946x over previous
import jax
import jax.numpy as jnp
from jax import lax
from jax.experimental import pallas as pl
from jax.experimental.pallas import tpu as pltpu

_EPS = 1e-12
_BS = 512


def _body(x_ref, t_ref, g_ref, b_ref, o_ref):
    emb = x_ref[...] + t_ref[...]
    mean = jnp.mean(emb, axis=-1, keepdims=True)
    cent = emb - mean
    var = jnp.mean(cent * cent, axis=-1, keepdims=True)
    o_ref[...] = cent * lax.rsqrt(var + _EPS) * g_ref[...] + b_ref[...]


def kernel(x, table, gamma, beta):
    B, S, D = x.shape
    bs = _BS if S % _BS == 0 else S
    grid = (S // bs, B)
    return pl.pallas_call(
        _body,
        grid=grid,
        in_specs=[
            pl.BlockSpec((1, bs, D), lambda i, b: (b, i, 0)),
            pl.BlockSpec((bs, D), lambda i, b: (i, 0)),
            pl.BlockSpec((1, D), lambda i, b: (0, 0)),
            pl.BlockSpec((1, D), lambda i, b: (0, 0)),
        ],
        out_specs=pl.BlockSpec((1, bs, D), lambda i, b: (b, i, 0)),
        out_shape=jax.ShapeDtypeStruct((B, S, D), x.dtype),
        compiler_params=pltpu.CompilerParams(
            dimension_semantics=("arbitrary", "arbitrary"),
        ),
    )(x, table[:S], gamma.reshape(1, D), beta.reshape(1, D))


# speedup vs baseline: 2.3653x; 1.1858x over previous
import jax
import jax.numpy as jnp
from jax import lax
from jax.experimental import pallas as pl
from jax.experimental.pallas import tpu as pltpu

_EPS = 1e-12
_BS = 1024


def _body(x_ref, t_ref, g_ref, b_ref, o_ref):
    emb = x_ref[...] + t_ref[...]
    mean = jnp.mean(emb, axis=-1, keepdims=True)
    cent = emb - mean
    var = jnp.mean(cent * cent, axis=-1, keepdims=True)
    o_ref[...] = cent * lax.rsqrt(var + _EPS) * g_ref[...] + b_ref[...]


def kernel(x, table, gamma, beta):
    B, S, D = x.shape
    bs = _BS if S % _BS == 0 else S
    grid = (S // bs, B)
    return pl.pallas_call(
        _body,
        grid=grid,
        in_specs=[
            pl.BlockSpec((1, bs, D), lambda i, b: (b, i, 0)),
            pl.BlockSpec((bs, D), lambda i, b: (i, 0)),
            pl.BlockSpec((1, D), lambda i, b: (0, 0)),
            pl.BlockSpec((1, D), lambda i, b: (0, 0)),
        ],
        out_specs=pl.BlockSpec((1, bs, D), lambda i, b: (b, i, 0)),
        out_shape=jax.ShapeDtypeStruct((B, S, D), x.dtype),
        compiler_params=pltpu.CompilerParams(
            dimension_semantics=("arbitrary", "arbitrary"),
        ),
    )(x, table[:S], gamma.reshape(1, D), beta.reshape(1, D))


# speedup vs baseline: 2.5472x; 1.0769x over previous
import jax
import jax.numpy as jnp
from jax import lax
from jax.experimental import pallas as pl
from jax.experimental.pallas import tpu as pltpu

_EPS = 1e-12
_BS = 2048


def _body(x_ref, t_ref, g_ref, b_ref, o_ref):
    emb = x_ref[...] + t_ref[...]
    mean = jnp.mean(emb, axis=-1, keepdims=True)
    cent = emb - mean
    var = jnp.mean(cent * cent, axis=-1, keepdims=True)
    o_ref[...] = cent * lax.rsqrt(var + _EPS) * g_ref[...] + b_ref[...]


def kernel(x, table, gamma, beta):
    B, S, D = x.shape
    bs = _BS if S % _BS == 0 else S
    grid = (S // bs, B)
    return pl.pallas_call(
        _body,
        grid=grid,
        in_specs=[
            pl.BlockSpec((1, bs, D), lambda i, b: (b, i, 0)),
            pl.BlockSpec((bs, D), lambda i, b: (i, 0)),
            pl.BlockSpec((1, D), lambda i, b: (0, 0)),
            pl.BlockSpec((1, D), lambda i, b: (0, 0)),
        ],
        out_specs=pl.BlockSpec((1, bs, D), lambda i, b: (b, i, 0)),
        out_shape=jax.ShapeDtypeStruct((B, S, D), x.dtype),
        compiler_params=pltpu.CompilerParams(
            dimension_semantics=("arbitrary", "arbitrary"),
        ),
    )(x, table[:S], gamma.reshape(1, D), beta.reshape(1, D))
